# baseline (speedup 1.0000x reference)
import jax
import jax.numpy as jnp
from jax.experimental import pallas as pl
from jax.experimental.pallas import tpu as pltpu

_BT = 256
_E = 64
_K = 8


def _gate_block(x_ref, wt_ref, wout_ref, iout_ref):
    x = x_ref[...]
    wt = wt_ref[...]
    scores = jax.lax.dot_general(
        x, wt, (((1,), (0,)), ((), ())),
        preferred_element_type=jnp.float32)
    m = jnp.max(scores, axis=1, keepdims=True)
    e = jnp.exp(scores - m)
    p = e / jnp.sum(e, axis=1, keepdims=True)

    lane = jax.lax.broadcasted_iota(jnp.int32, (_BT, _E), 1)
    vals = []
    idxs = []
    for _ in range(_K):
        v = jnp.max(p, axis=1, keepdims=True)
        hit = p >= v
        idx = jnp.min(jnp.where(hit, lane, _E), axis=1, keepdims=True)
        vals.append(v)
        idxs.append(idx)
        p = jnp.where(lane == idx, -1.0, p)
    wout_ref[...] = jnp.concatenate(vals, axis=1)
    iout_ref[...] = jnp.concatenate(idxs, axis=1)


def kernel(x, weight):
    t = x.shape[0]
    wt = weight.T
    grid = (t // _BT,)
    wout, iout = pl.pallas_call(
        _gate_block,
        grid=grid,
        in_specs=[
            pl.BlockSpec((_BT, x.shape[1]), lambda i: (i, 0)),
            pl.BlockSpec((x.shape[1], _E), lambda i: (0, 0)),
        ],
        out_specs=[
            pl.BlockSpec((_BT, _K), lambda i: (i, 0)),
            pl.BlockSpec((_BT, _K), lambda i: (i, 0)),
        ],
        out_shape=[
            jax.ShapeDtypeStruct((t, _K), jnp.float32),
            jax.ShapeDtypeStruct((t, _K), jnp.int32),
        ],
    )(x, wt)
    return wout, iout
